# baseline (speedup 1.0000x reference)
import functools

import jax
import jax.numpy as jnp
from jax.experimental import pallas as pl
from jax.experimental.pallas import tpu as pltpu

_K = 20
_NEG_LARGE = -jnp.inf


def _layer1_kernel(x_cn_ref, xt_ref, nt_ref, w_ref,
                   rowmax_ref, rowmin_ref, sum_ref, sq_ref,
                   nd_scr, hsum_scr, hsq_scr):
    x_cn = x_cn_ref[0]
    xt = xt_ref[0]
    nt = nt_ref[0]
    w = w_ref[...]
    n_pts = xt.shape[0]

    xx_row = jnp.sum(x_cn * x_cn, axis=0, keepdims=True)
    xx_col = jnp.sum(xt * xt, axis=1, keepdims=True)
    inner = jax.lax.dot_general(
        xt, xt, (((1,), (1,)), ((), ())),
        preferred_element_type=jnp.float32)
    nd_scr[...] = -(xx_col - 2.0 * inner + xx_row)

    cat = jnp.concatenate([xt, nt], axis=1)

    rowmax_ref[0] = jnp.full((n_pts, w.shape[0]), -jnp.inf, jnp.float32)
    rowmin_ref[0] = jnp.full((n_pts, w.shape[0]), jnp.inf, jnp.float32)
    hsum_scr[...] = jnp.zeros_like(hsum_scr)
    hsq_scr[...] = jnp.zeros_like(hsq_scr)

    def body(_, carry):
        cur = nd_scr[...]
        lane = jax.lax.broadcasted_iota(jnp.int32, cur.shape, 1)
        jidx = jnp.argmax(cur, axis=1)[:, None]
        onehot_b = lane == jidx
        onehot = onehot_b.astype(jnp.float32)
        sel = jax.lax.dot_general(
            onehot, cat, (((1,), (0,)), ((), ())),
            preferred_element_type=jnp.float32)
        pj = sel[:, 0:3]
        nj = sel[:, 3:6]
        d = pj - xt
        dist = jnp.sqrt(jnp.sum(d * d, axis=1, keepdims=True) + 1e-12)
        ang = jnp.sum(nt * nj, axis=1, keepdims=True)
        feat = jnp.concatenate([xt, d, nt, nj, dist, ang], axis=1)
        h = jax.lax.dot_general(
            feat, w, (((1,), (1,)), ((), ())),
            preferred_element_type=jnp.float32)
        rowmax_ref[0] = jnp.maximum(rowmax_ref[0], h)
        rowmin_ref[0] = jnp.minimum(rowmin_ref[0], h)
        hsum_scr[...] = hsum_scr[...] + h
        hsq_scr[...] = hsq_scr[...] + h * h
        nd_scr[...] = jnp.where(onehot_b, _NEG_LARGE, cur)
        return carry

    jax.lax.fori_loop(0, _K, body, 0)

    sum_ref[0] = jnp.sum(hsum_scr[...], axis=0, keepdims=True)
    sq_ref[0] = jnp.sum(hsq_scr[...], axis=0, keepdims=True)


def _bn_kernel(rowmax_ref, rowmin_ref, sums_ref, sqs_ref, g_ref, b_ref,
               out_ref, *, count):
    sums = jnp.sum(sums_ref[...], axis=(0, 1), keepdims=False)[None, :]
    sqs = jnp.sum(sqs_ref[...], axis=(0, 1), keepdims=False)[None, :]
    mean = sums / count
    var = sqs / count - mean * mean
    scale = g_ref[...] / jnp.sqrt(var + 1e-5)
    shift = b_ref[...] - mean * scale
    hext = jnp.where(scale >= 0.0, rowmax_ref[0], rowmin_ref[0])
    y = hext * scale + shift
    out_ref[0] = jnp.where(y >= 0.0, y, 0.2 * y)


def kernel(x, n, W1, g1, b1, W2, g2, b2, W3, g3, b3, W4, g4, b4):
    del W2, g2, b2, W3, g3, b3, W4, g4, b4
    B, _, N = x.shape
    C = W1.shape[0]
    xt = jnp.transpose(x, (0, 2, 1))
    nt = jnp.transpose(n, (0, 2, 1))

    rowmax, rowmin, sums, sqs = pl.pallas_call(
        _layer1_kernel,
        grid=(B,),
        in_specs=[
            pl.BlockSpec((1, 3, N), lambda b: (b, 0, 0)),
            pl.BlockSpec((1, N, 3), lambda b: (b, 0, 0)),
            pl.BlockSpec((1, N, 3), lambda b: (b, 0, 0)),
            pl.BlockSpec((C, 14), lambda b: (0, 0)),
        ],
        out_specs=[
            pl.BlockSpec((1, N, C), lambda b: (b, 0, 0)),
            pl.BlockSpec((1, N, C), lambda b: (b, 0, 0)),
            pl.BlockSpec((1, 1, C), lambda b: (b, 0, 0)),
            pl.BlockSpec((1, 1, C), lambda b: (b, 0, 0)),
        ],
        out_shape=[
            jax.ShapeDtypeStruct((B, N, C), jnp.float32),
            jax.ShapeDtypeStruct((B, N, C), jnp.float32),
            jax.ShapeDtypeStruct((B, 1, C), jnp.float32),
            jax.ShapeDtypeStruct((B, 1, C), jnp.float32),
        ],
        scratch_shapes=[
            pltpu.VMEM((N, N), jnp.float32),
            pltpu.VMEM((N, C), jnp.float32),
            pltpu.VMEM((N, C), jnp.float32),
        ],
    )(x, xt, nt, W1)

    count = float(B * N * _K)
    x1t = pl.pallas_call(
        functools.partial(_bn_kernel, count=count),
        grid=(B,),
        in_specs=[
            pl.BlockSpec((1, N, C), lambda b: (b, 0, 0)),
            pl.BlockSpec((1, N, C), lambda b: (b, 0, 0)),
            pl.BlockSpec((B, 1, C), lambda b: (0, 0, 0)),
            pl.BlockSpec((B, 1, C), lambda b: (0, 0, 0)),
            pl.BlockSpec((1, C), lambda b: (0, 0)),
            pl.BlockSpec((1, C), lambda b: (0, 0)),
        ],
        out_specs=pl.BlockSpec((1, N, C), lambda b: (b, 0, 0)),
        out_shape=jax.ShapeDtypeStruct((B, N, C), jnp.float32),
    )(rowmax, rowmin, sums, sqs, g1.reshape(1, C), b1.reshape(1, C))

    return (x1t, x)


# speedup vs baseline: 1.0993x; 1.0993x over previous
import functools

import jax
import jax.numpy as jnp
from jax.experimental import pallas as pl
from jax.experimental.pallas import tpu as pltpu

_K = 20
_NEG_LARGE = -jnp.inf


def _layer1_kernel(x_cn_ref, xt_ref, nt_ref, w_ref,
                   rowmax_ref, rowmin_ref, sum_ref, sq_ref,
                   nd_scr, hsum_scr, hsq_scr):
    x_cn = x_cn_ref[0]
    xt = xt_ref[0]
    nt = nt_ref[0]
    w = w_ref[...]
    n_pts = xt.shape[0]

    xx_row = jnp.sum(x_cn * x_cn, axis=0, keepdims=True)
    xx_col = jnp.sum(xt * xt, axis=1, keepdims=True)
    inner = jax.lax.dot_general(
        xt, xt, (((1,), (1,)), ((), ())),
        preferred_element_type=jnp.float32)
    nd = -(xx_col - 2.0 * inner + xx_row)
    bits = jax.lax.bitcast_convert_type(nd, jnp.int32)
    key = jnp.where(bits < 0, bits ^ jnp.int32(0x7FFFFFFF), bits)
    lane0 = jax.lax.broadcasted_iota(jnp.int32, nd.shape, 1)
    nd_scr[...] = (key & jnp.int32(~1023)) | (jnp.int32(1023) - lane0)

    cat = jnp.concatenate([xt, nt], axis=1)

    rowmax_ref[0] = jnp.full((n_pts, w.shape[0]), -jnp.inf, jnp.float32)
    rowmin_ref[0] = jnp.full((n_pts, w.shape[0]), jnp.inf, jnp.float32)
    hsum_scr[...] = jnp.zeros_like(hsum_scr)
    hsq_scr[...] = jnp.zeros_like(hsq_scr)

    def body(_, carry):
        cur = nd_scr[...]
        m = jnp.max(cur, axis=1, keepdims=True)
        onehot_b = cur == m
        onehot = onehot_b.astype(jnp.float32)
        sel = jax.lax.dot_general(
            onehot, cat, (((1,), (0,)), ((), ())),
            preferred_element_type=jnp.float32)
        pj = sel[:, 0:3]
        nj = sel[:, 3:6]
        d = pj - xt
        dist = jnp.sqrt(jnp.sum(d * d, axis=1, keepdims=True) + 1e-12)
        ang = jnp.sum(nt * nj, axis=1, keepdims=True)
        feat = jnp.concatenate([xt, d, nt, nj, dist, ang], axis=1)
        h = jax.lax.dot_general(
            feat, w, (((1,), (1,)), ((), ())),
            preferred_element_type=jnp.float32)
        rowmax_ref[0] = jnp.maximum(rowmax_ref[0], h)
        rowmin_ref[0] = jnp.minimum(rowmin_ref[0], h)
        hsum_scr[...] = hsum_scr[...] + h
        hsq_scr[...] = hsq_scr[...] + h * h
        nd_scr[...] = jnp.where(onehot_b, jnp.int32(-2147483648), cur)
        return carry

    jax.lax.fori_loop(0, _K, body, 0)

    sum_ref[0] = jnp.sum(hsum_scr[...], axis=0, keepdims=True)
    sq_ref[0] = jnp.sum(hsq_scr[...], axis=0, keepdims=True)


def _bn_kernel(rowmax_ref, rowmin_ref, sums_ref, sqs_ref, g_ref, b_ref,
               out_ref, *, count):
    sums = jnp.sum(sums_ref[...], axis=(0, 1), keepdims=False)[None, :]
    sqs = jnp.sum(sqs_ref[...], axis=(0, 1), keepdims=False)[None, :]
    mean = sums / count
    var = sqs / count - mean * mean
    scale = g_ref[...] / jnp.sqrt(var + 1e-5)
    shift = b_ref[...] - mean * scale
    hext = jnp.where(scale >= 0.0, rowmax_ref[0], rowmin_ref[0])
    y = hext * scale + shift
    out_ref[0] = jnp.where(y >= 0.0, y, 0.2 * y)


def kernel(x, n, W1, g1, b1, W2, g2, b2, W3, g3, b3, W4, g4, b4):
    del W2, g2, b2, W3, g3, b3, W4, g4, b4
    B, _, N = x.shape
    C = W1.shape[0]
    xt = jnp.transpose(x, (0, 2, 1))
    nt = jnp.transpose(n, (0, 2, 1))

    rowmax, rowmin, sums, sqs = pl.pallas_call(
        _layer1_kernel,
        grid=(B,),
        in_specs=[
            pl.BlockSpec((1, 3, N), lambda b: (b, 0, 0)),
            pl.BlockSpec((1, N, 3), lambda b: (b, 0, 0)),
            pl.BlockSpec((1, N, 3), lambda b: (b, 0, 0)),
            pl.BlockSpec((C, 14), lambda b: (0, 0)),
        ],
        out_specs=[
            pl.BlockSpec((1, N, C), lambda b: (b, 0, 0)),
            pl.BlockSpec((1, N, C), lambda b: (b, 0, 0)),
            pl.BlockSpec((1, 1, C), lambda b: (b, 0, 0)),
            pl.BlockSpec((1, 1, C), lambda b: (b, 0, 0)),
        ],
        out_shape=[
            jax.ShapeDtypeStruct((B, N, C), jnp.float32),
            jax.ShapeDtypeStruct((B, N, C), jnp.float32),
            jax.ShapeDtypeStruct((B, 1, C), jnp.float32),
            jax.ShapeDtypeStruct((B, 1, C), jnp.float32),
        ],
        scratch_shapes=[
            pltpu.VMEM((N, N), jnp.int32),
            pltpu.VMEM((N, C), jnp.float32),
            pltpu.VMEM((N, C), jnp.float32),
        ],
    )(x, xt, nt, W1)

    count = float(B * N * _K)
    x1t = pl.pallas_call(
        functools.partial(_bn_kernel, count=count),
        grid=(B,),
        in_specs=[
            pl.BlockSpec((1, N, C), lambda b: (b, 0, 0)),
            pl.BlockSpec((1, N, C), lambda b: (b, 0, 0)),
            pl.BlockSpec((B, 1, C), lambda b: (0, 0, 0)),
            pl.BlockSpec((B, 1, C), lambda b: (0, 0, 0)),
            pl.BlockSpec((1, C), lambda b: (0, 0)),
            pl.BlockSpec((1, C), lambda b: (0, 0)),
        ],
        out_specs=pl.BlockSpec((1, N, C), lambda b: (b, 0, 0)),
        out_shape=jax.ShapeDtypeStruct((B, N, C), jnp.float32),
    )(rowmax, rowmin, sums, sqs, g1.reshape(1, C), b1.reshape(1, C))

    return (x1t, x)
